# baseline (speedup 1.0000x reference)
import jax
import jax.numpy as jnp
from jax import lax
from jax.experimental import pallas as pl
from jax.experimental.pallas import tpu as pltpu
from jax.experimental.pallas import tpu_sc as plsc

BATCH = 16384
FIELDS = 26
FEATURES = 64

NC = 2
NS = 16
NW = NC * NS
CH = 128
NBUF = 4

NUM_ROWS2 = 500000
B = BATCH * FIELDS
BPW = B // NW
NCHUNK = BPW // CH
NGROUP = NCHUNK // NBUF


def _embed_body(idx_hbm, table_hbm, out_hbm, idx_v, rows_v, *sems):
    wid = lax.axis_index("s") * NC + lax.axis_index("c")
    base = wid * BPW

    pltpu.sync_copy(idx_hbm.at[wid], idx_v)

    for b in range(NBUF):
        pltpu.async_copy(table_hbm.at[idx_v.at[b]], rows_v.at[b], sems[b])

    def group(g, carry):
        j = g * NBUF
        for b in range(NBUF):
            chunk = j + b
            pltpu.make_async_copy(
                table_hbm.at[idx_v.at[chunk]], rows_v.at[b], sems[b]
            ).wait()
            pltpu.sync_copy(
                rows_v.at[b], out_hbm.at[pl.ds(base + chunk * CH, CH)]
            )

            @pl.when(chunk + NBUF < NCHUNK)
            def _():
                pltpu.async_copy(
                    table_hbm.at[idx_v.at[chunk + NBUF]], rows_v.at[b], sems[b]
                )

        return carry

    lax.fori_loop(0, NGROUP, group, 0)


@jax.jit
def _run(idx2, table2):
    f = pl.kernel(
        _embed_body,
        out_type=jax.ShapeDtypeStruct((B, 2 * FEATURES), jnp.float32),
        mesh=plsc.VectorSubcoreMesh(core_axis_name="c", subcore_axis_name="s"),
        scratch_types=[
            pltpu.VMEM((NCHUNK, CH), jnp.int32),
            pltpu.VMEM((NBUF, CH, 2 * FEATURES), jnp.float32),
        ]
        + [pltpu.SemaphoreType.DMA] * NBUF,
    )
    return f(idx2, table2)


def kernel(inputs, embedding):
    flat = inputs.astype(jnp.int32).reshape(-1)
    idx2 = (flat // 2).reshape(NW, NCHUNK, CH)
    table2 = embedding.reshape(NUM_ROWS2, 2 * FEATURES)
    pairs = _run(idx2, table2)
    sel = (flat % 2).astype(bool)[:, None]
    out = jnp.where(sel, pairs[:, FEATURES:], pairs[:, :FEATURES])
    return out.reshape(BATCH, FIELDS, FEATURES)


# speedup vs baseline: 1.8856x; 1.8856x over previous
import jax
import jax.numpy as jnp
from jax import lax
from jax.experimental import pallas as pl
from jax.experimental.pallas import tpu as pltpu
from jax.experimental.pallas import tpu_sc as plsc

BATCH = 16384
FIELDS = 26
FEATURES = 64

NC = 2
NS = 16
NW = NC * NS
BPW_B = BATCH // NW
CB = 8
CH = CB * FIELDS
NCHUNK = BPW_B // CB
NBUF = 2


def _embed_body(idx_hbm, table_hbm, out_hbm, idx_v, rows_v, *sems):
    wid = lax.axis_index("s") * NC + lax.axis_index("c")
    b0 = wid * BPW_B

    pltpu.sync_copy(idx_hbm.at[wid], idx_v)

    def issue(c, b):
        for g in range(CH // 16):
            iv = idx_v[c, pl.ds(g * 16, 16)]
            for l in range(16):
                pltpu.async_copy(
                    table_hbm.at[pl.ds(iv[l], 1)],
                    rows_v.at[b, pl.ds(g * 16 + l, 1)],
                    sems[b],
                )

    def drain(b):
        pltpu.make_async_copy(
            table_hbm.at[pl.ds(0, CH)], rows_v.at[b], sems[b]
        ).wait()

    issue(0, 0)

    def group(g, carry):
        c0 = g * NBUF
        for bb in range(NBUF):
            c = c0 + bb
            drain(bb)

            @pl.when(c + 1 < NCHUNK)
            def _():
                issue(c + 1, (bb + 1) % NBUF)

            for k in range(CB):
                pltpu.sync_copy(
                    rows_v.at[bb, pl.ds(k * FIELDS, FIELDS)],
                    out_hbm.at[b0 + c * CB + k],
                )
        return carry

    lax.fori_loop(0, NCHUNK // NBUF, group, 0)


@jax.jit
def _run(idx, table):
    f = pl.kernel(
        _embed_body,
        out_type=jax.ShapeDtypeStruct((BATCH, FIELDS, FEATURES), jnp.float32),
        mesh=plsc.VectorSubcoreMesh(core_axis_name="c", subcore_axis_name="s"),
        scratch_types=[
            pltpu.VMEM((NCHUNK, CH), jnp.int32),
            pltpu.VMEM((NBUF, CH, FEATURES), jnp.float32),
        ]
        + [pltpu.SemaphoreType.DMA] * NBUF,
    )
    return f(idx, table)


def kernel(inputs, embedding):
    idx = inputs.astype(jnp.int32).reshape(NW, NCHUNK, CH)
    return _run(idx, embedding)


# speedup vs baseline: 1.8864x; 1.0004x over previous
import jax
import jax.numpy as jnp
from jax import lax
from jax.experimental import pallas as pl
from jax.experimental.pallas import tpu as pltpu
from jax.experimental.pallas import tpu_sc as plsc

BATCH = 16384
FIELDS = 26
FEATURES = 64

NC = 2
NS = 16
NW = NC * NS
BPW_B = BATCH // NW
CB = 8
CH = CB * FIELDS
NCHUNK = BPW_B // CB
NBUF = 2


def _embed_body(idx_hbm, table_hbm, out_hbm, idx_v, rows_v, *sems):
    wid = lax.axis_index("s") * NC + lax.axis_index("c")
    b0 = wid * BPW_B

    pltpu.sync_copy(idx_hbm.at[wid], idx_v)

    def issue(c, b):
        for g in range(CH // 16):
            iv = idx_v[c, pl.ds(g * 16, 16)]
            for l in range(16):
                pltpu.async_copy(
                    table_hbm.at[pl.ds(iv[l], 1)],
                    rows_v.at[b, pl.ds(g * 16 + l, 1)],
                    sems[b],
                )

    def drain(b):
        pltpu.make_async_copy(
            table_hbm.at[pl.ds(0, CH)], rows_v.at[b], sems[b]
        ).wait()

    issue(0, 0)

    def group(g, carry):
        c0 = g * NBUF
        for bb in range(NBUF):
            c = c0 + bb
            drain(bb)

            @pl.when(c + 1 < NCHUNK)
            def _():
                issue(c + 1, (bb + 1) % NBUF)

            for k in range(CB):
                pltpu.sync_copy(
                    rows_v.at[bb, pl.ds(k * FIELDS, FIELDS)],
                    out_hbm.at[b0 + c * CB + k],
                )
        return carry

    lax.fori_loop(0, NCHUNK // NBUF, group, 0)


@jax.jit
def _run(idx, table):
    f = pl.kernel(
        _embed_body,
        out_type=jax.ShapeDtypeStruct((BATCH, FIELDS, FEATURES), jnp.float32),
        mesh=plsc.VectorSubcoreMesh(core_axis_name="c", subcore_axis_name="s"),
        scratch_types=[
            pltpu.VMEM((NCHUNK, CH), jnp.int32),
            pltpu.VMEM((NBUF, CH, FEATURES), jnp.float32),
        ]
        + [pltpu.SemaphoreType.DMA] * NBUF,
        compiler_params=pltpu.CompilerParams(skip_device_barrier=True),
    )
    return f(idx, table)


def kernel(inputs, embedding):
    idx = inputs.astype(jnp.int32).reshape(NW, NCHUNK, CH)
    return _run(idx, embedding)


# speedup vs baseline: 1.8914x; 1.0027x over previous
import jax
import jax.numpy as jnp
from jax import lax
from jax.experimental import pallas as pl
from jax.experimental.pallas import tpu as pltpu
from jax.experimental.pallas import tpu_sc as plsc

BATCH = 16384
FIELDS = 26
FEATURES = 64

NC = 2
NS = 16
NW = NC * NS
BPW_B = BATCH // NW
CB = 8
CH = CB * FIELDS
NCHUNK = BPW_B // CB
NBUF = 4


def _embed_body(idx_hbm, table_hbm, out_hbm, idx_v, rows_v, *sems):
    gsems = sems[:NBUF]
    wsems = sems[NBUF:]
    wid = lax.axis_index("s") * NC + lax.axis_index("c")
    b0 = wid * BPW_B

    pltpu.sync_copy(idx_hbm.at[wid], idx_v)

    def issue(c, b):
        for g in range(CH // 16):
            iv = idx_v[c, pl.ds(g * 16, 16)]
            for l in range(16):
                pltpu.async_copy(
                    table_hbm.at[pl.ds(iv[l], 1)],
                    rows_v.at[b, pl.ds(g * 16 + l, 1)],
                    gsems[b],
                )

    def drain(b):
        pltpu.make_async_copy(
            table_hbm.at[pl.ds(0, CH)], rows_v.at[b], gsems[b]
        ).wait()

    def wait_write(b):
        pltpu.make_async_copy(
            out_hbm.at[pl.ds(0, CB)], out_hbm.at[pl.ds(0, CB)], wsems[b]
        ).wait()

    issue(0, 0)

    def group(g, carry):
        for bb in range(NBUF):
            c = g * NBUF + bb
            nb = (bb + 1) % NBUF
            drain(bb)

            @pl.when(c + 1 < NCHUNK)
            def _():
                @pl.when(c >= NBUF - 1)
                def _():
                    wait_write(nb)

                issue(c + 1, nb)

            for k in range(CB):
                pltpu.async_copy(
                    rows_v.at[bb, pl.ds(k * FIELDS, FIELDS)],
                    out_hbm.at[b0 + c * CB + k],
                    wsems[bb],
                )
        return carry

    lax.fori_loop(0, NCHUNK // NBUF, group, 0)

    for bb in range(NBUF):
        wait_write(bb)


@jax.jit
def _run(idx, table):
    f = pl.kernel(
        _embed_body,
        out_type=jax.ShapeDtypeStruct((BATCH, FIELDS, FEATURES), jnp.float32),
        mesh=plsc.VectorSubcoreMesh(core_axis_name="c", subcore_axis_name="s"),
        scratch_types=[
            pltpu.VMEM((NCHUNK, CH), jnp.int32),
            pltpu.VMEM((NBUF, CH, FEATURES), jnp.float32),
        ]
        + [pltpu.SemaphoreType.DMA] * (2 * NBUF),
    )
    return f(idx, table)


def kernel(inputs, embedding):
    idx = inputs.astype(jnp.int32).reshape(NW, NCHUNK, CH)
    return _run(idx, embedding)
